# Initial kernel scaffold; baseline (speedup 1.0000x reference)
import jax
import jax.numpy as jnp
from jax.experimental import pallas as pl


def kernel(pred_logits, pred_boxes, tgt_boxes, tgt_labels, src_idx):
    raise NotImplementedError("write your pallas kernel here")


# speedup vs baseline: 1.9511x; 1.9511x over previous
import functools

import jax
import jax.numpy as jnp
from jax.experimental import pallas as pl
from jax.experimental.pallas import tpu as pltpu

ALPHA = 0.25
GAMMA = 2.0
W_CE = 2.0
W_BBOX = 5.0
W_GIOU = 2.0


def _xyxy_cols(bx):
    cx = bx[:, 0:1]
    cy = bx[:, 1:2]
    w = bx[:, 2:3]
    h = bx[:, 3:4]
    return cx - 0.5 * w, cy - 0.5 * h, cx + 0.5 * w, cy + 0.5 * h


def _loss_kernel(logits_ref, boxes_ref, tboxes_ref, lab_col_ref, sidx_col_ref,
                 sidx_row_ref, out_ref, acc_ref, *, nb, nbatch):
    b = pl.program_id(0)

    @pl.when(b == 0)
    def _init():
        acc_ref[0] = 0.0
        acc_ref[1] = 0.0
        acc_ref[2] = 0.0

    x = logits_ref[0]
    boxes = boxes_ref[0]
    tb = tboxes_ref[0]
    lab_col = lab_col_ref[0]
    sidx_col = sidx_col_ref[0]
    sidx_row = sidx_row_ref[0]

    q = x.shape[0]
    c = x.shape[1]
    t = tb.shape[0]

    iota_q1 = jax.lax.broadcasted_iota(jnp.int32, (t, q), 1)
    oh_tq = (sidx_col == iota_q1).astype(jnp.float32)
    iota_c1 = jax.lax.broadcasted_iota(jnp.int32, (t, c), 1)
    oh_tc = (lab_col == iota_c1).astype(jnp.float32)

    oh_qt = (jnp.broadcast_to(sidx_row, (q, t)) ==
             jax.lax.broadcasted_iota(jnp.int32, (q, t), 0)).astype(jnp.float32)
    cnt = jax.lax.dot_general(oh_qt, oh_tc, (((1,), (0,)), ((), ())),
                              preferred_element_type=jnp.float32)
    tgt = jnp.minimum(cnt, 1.0)

    prob = jax.nn.sigmoid(x)
    ce = jnp.maximum(x, 0.0) - x * tgt + jnp.log1p(jnp.exp(-jnp.abs(x)))
    p_t = prob * tgt + (1.0 - prob) * (1.0 - tgt)
    alpha_t = ALPHA * tgt + (1.0 - ALPHA) * (1.0 - tgt)
    one_m = 1.0 - p_t
    ce_part = jnp.sum(alpha_t * one_m * one_m * ce)

    pb = jax.lax.dot_general(oh_tq, boxes, (((1,), (0,)), ((), ())),
                             preferred_element_type=jnp.float32,
                             precision=jax.lax.Precision.HIGHEST)

    l1_part = jnp.sum(jnp.abs(pb - tb))

    px1, py1, px2, py2 = _xyxy_cols(pb)
    tx1, ty1, tx2, ty2 = _xyxy_cols(tb)
    area_p = (px2 - px1) * (py2 - py1)
    area_t = (tx2 - tx1) * (ty2 - ty1)
    iw = jnp.clip(jnp.minimum(px2, tx2) - jnp.maximum(px1, tx1), 0.0, None)
    ih = jnp.clip(jnp.minimum(py2, ty2) - jnp.maximum(py1, ty1), 0.0, None)
    inter = iw * ih
    union = area_p + area_t - inter
    iou = inter / union
    ew = jnp.clip(jnp.maximum(px2, tx2) - jnp.minimum(px1, tx1), 0.0, None)
    eh = jnp.clip(jnp.maximum(py2, ty2) - jnp.minimum(py1, ty1), 0.0, None)
    earea = ew * eh
    g = iou - (earea - union) / earea
    giou_part = jnp.sum(1.0 - g)

    acc_ref[0] += ce_part
    acc_ref[1] += l1_part
    acc_ref[2] += giou_part

    @pl.when(b == nbatch - 1)
    def _finish():
        ce_l = acc_ref[0] / nb
        bb_l = acc_ref[1] / nb
        gi_l = acc_ref[2] / nb
        out_ref[0] = ce_l
        out_ref[1] = bb_l
        out_ref[2] = gi_l
        out_ref[3] = W_CE * ce_l + W_BBOX * bb_l + W_GIOU * gi_l


def kernel(pred_logits, pred_boxes, tgt_boxes, tgt_labels, src_idx):
    B, Q, C = pred_logits.shape
    T = tgt_labels.shape[1]
    nb = float(max(1, B * T))

    lab_col = tgt_labels.reshape(B, T, 1).astype(jnp.int32)
    sidx_col = src_idx.reshape(B, T, 1).astype(jnp.int32)
    sidx_row = src_idx.reshape(B, 1, T).astype(jnp.int32)

    out = pl.pallas_call(
        functools.partial(_loss_kernel, nb=nb, nbatch=B),
        grid=(B,),
        in_specs=[
            pl.BlockSpec((1, Q, C), lambda b: (b, 0, 0)),
            pl.BlockSpec((1, Q, 4), lambda b: (b, 0, 0)),
            pl.BlockSpec((1, T, 4), lambda b: (b, 0, 0)),
            pl.BlockSpec((1, T, 1), lambda b: (b, 0, 0)),
            pl.BlockSpec((1, T, 1), lambda b: (b, 0, 0)),
            pl.BlockSpec((1, 1, T), lambda b: (b, 0, 0)),
        ],
        out_specs=pl.BlockSpec(memory_space=pltpu.SMEM),
        out_shape=jax.ShapeDtypeStruct((4,), jnp.float32),
        scratch_shapes=[pltpu.SMEM((3,), jnp.float32)],
        compiler_params=pltpu.CompilerParams(
            dimension_semantics=("arbitrary",),
        ),
    )(pred_logits, pred_boxes, tgt_boxes, lab_col, sidx_col, sidx_row)

    return (out[0], out[1], out[2], out[3])
